# baseline (speedup 1.0000x reference)
import functools

import jax
import jax.numpy as jnp
from jax import lax
from jax.experimental import pallas as pl
from jax.experimental.pallas import tpu as pltpu, tpu_sc as plsc

NUM_CORES = 2
NUM_SUBCORES = 16
NW = NUM_CORES * NUM_SUBCORES
CHUNK = 128


def _embed(labels2d, table, b_per_w, n_chunks, D):
    mesh = plsc.VectorSubcoreMesh(core_axis_name="c", subcore_axis_name="s")

    @functools.partial(
        pl.kernel,
        out_type=jax.ShapeDtypeStruct((NW, b_per_w, D), jnp.float32),
        mesh=mesh,
        compiler_params=pltpu.CompilerParams(use_tc_tiling_on_sc=False),
        scratch_types=[
            pltpu.VMEM((n_chunks, CHUNK), jnp.int32),
            pltpu.VMEM((b_per_w, D), jnp.float32),
            pltpu.SemaphoreType.DMA,
        ],
    )
    def k(table_hbm, idx_hbm, out_hbm, idx_v, rows_v, sem):
        wid = lax.axis_index("s") * NUM_CORES + lax.axis_index("c")
        pltpu.sync_copy(idx_hbm.at[wid], idx_v)
        copies = []
        for j in range(n_chunks):
            copies.append(
                pltpu.async_copy(
                    table_hbm.at[idx_v.at[j]],
                    rows_v.at[pl.ds(j * CHUNK, CHUNK)],
                    sem,
                )
            )
        for c in copies:
            c.wait()
        pltpu.sync_copy(rows_v, out_hbm.at[wid])

    return k(table, labels2d)


def kernel(labels, train, table):
    B = labels.shape[0]
    V, D = table.shape
    b_per_w = B // NW
    n_chunks = b_per_w // CHUNK
    labels2d = labels.astype(jnp.int32).reshape(NW, n_chunks, CHUNK)
    out = _embed(labels2d, table, b_per_w, n_chunks, D)
    return out.reshape(B, D)


# speedup vs baseline: 1.7066x; 1.7066x over previous
import functools

import jax
import jax.numpy as jnp
from jax import lax
from jax.experimental import pallas as pl
from jax.experimental.pallas import tpu as pltpu, tpu_sc as plsc

NUM_CORES = 2
NUM_SUBCORES = 16
NW = NUM_CORES * NUM_SUBCORES
L = 16


def _embed(labels2d, table, b_per_w, D):
    mesh = plsc.VectorSubcoreMesh(core_axis_name="c", subcore_axis_name="s")
    n_groups = b_per_w // L

    @functools.partial(
        pl.kernel,
        out_type=jax.ShapeDtypeStruct((NW, b_per_w, D), jnp.float32),
        mesh=mesh,
        scratch_types=[
            pltpu.VMEM((b_per_w,), jnp.int32),
            pltpu.VMEM((b_per_w, D), jnp.float32),
            pltpu.SemaphoreType.DMA,
        ],
    )
    def k(table_hbm, idx_hbm, out_hbm, idx_v, rows_v, sem):
        wid = lax.axis_index("s") * NUM_CORES + lax.axis_index("c")
        pltpu.sync_copy(idx_hbm.at[wid], idx_v)

        def group(g, _):
            vec = idx_v[pl.ds(g * L, L)]
            for l in range(L):
                r = jnp.squeeze(lax.slice(vec, (l,), (l + 1,)))
                pltpu.async_copy(table_hbm.at[r], rows_v.at[g * L + l], sem)
            return 0

        lax.fori_loop(0, n_groups, group, 0)
        pltpu.make_async_copy(out_hbm.at[wid], rows_v, sem).wait()
        pltpu.sync_copy(rows_v, out_hbm.at[wid])

    return k(table, labels2d)


def kernel(labels, train, table):
    B = labels.shape[0]
    V, D = table.shape
    b_per_w = B // NW
    labels2d = labels.astype(jnp.int32).reshape(NW, b_per_w)
    out = _embed(labels2d, table, b_per_w, D)
    return out.reshape(B, D)
